# baseline (speedup 1.0000x reference)
import functools
import jax
import jax.numpy as jnp
from jax import lax
from jax.experimental import pallas as pl
from jax.experimental.pallas import tpu as pltpu
from jax.experimental.pallas import tpu_sc as plsc

_NEG_INF = float("-inf")
_BIG = 2147483647

_RPB = 8
_W = 2000


def _topk_kernel(lp_ref, sums_ref, seq_ref,
                 seq_out_ref, ys_out_ref, bix_out_ref, scr_ref):
    R = lp_ref.shape[1]
    C = lp_ref.shape[2]
    KT = seq_ref.shape[2]
    T = 2
    K = KT // T
    cand = lp_ref[0] + sums_ref[0]
    scr_ref[...] = cand
    rowmax = jnp.max(cand, axis=1, keepdims=True)
    riota = lax.broadcasted_iota(jnp.int32, (R, 1), 0)
    ciota = lax.broadcasted_iota(jnp.int32, (1, C), 1)
    i8 = lax.broadcasted_iota(jnp.int32, (1, 8), 1)
    i16 = lax.broadcasted_iota(jnp.int32, (1, 16), 1)
    i10 = lax.broadcasted_iota(jnp.int32, (1, KT), 1)
    seq_row = seq_ref[0]
    ys_row = jnp.zeros((1, 8), jnp.float32)
    bix_row = jnp.zeros((1, 8), jnp.int32)
    seq_out_row = jnp.zeros((1, 16), jnp.int32)
    for k in range(K):
        m = jnp.max(rowmax)
        r = jnp.min(jnp.where(rowmax == m, riota, _BIG))
        row = scr_ref[pl.ds(r, 1), :]
        c = jnp.min(jnp.where(row == m, ciota, _BIG))
        bix = r // _RPB
        six = (r - bix * _RPB) * C + c
        masked = jnp.where(ciota == c, _NEG_INF, row)
        scr_ref[pl.ds(r, 1), :] = masked
        rowmax = jnp.where(riota == r, jnp.max(masked), rowmax)
        ys_row = jnp.where(i8 == k, m, ys_row)
        bix_row = jnp.where(i8 == k, bix, bix_row)
        for t in range(T):
            val = jnp.sum(jnp.where(i10 == bix * T + t, seq_row, 0))
            seq_out_row = jnp.where(i16 == k * (T + 1) + t, val, seq_out_row)
        seq_out_row = jnp.where(i16 == k * (T + 1) + T, six, seq_out_row)
    seq_out_ref[0] = seq_out_row
    ys_out_ref[0] = ys_row
    bix_out_ref[0] = bix_row


def _run_topk(logprobs, beam_logprobs_sum, beam_seq):
    B, K = beam_logprobs_sum.shape
    V = logprobs.shape[-1]
    T = beam_seq.shape[-1]
    R = K * _RPB
    C = V // _RPB
    lp40 = logprobs.reshape(B, R, C)
    sums40 = jnp.broadcast_to(beam_logprobs_sum[:, :, None],
                              (B, K, _RPB)).reshape(B, R, 1)
    seq3 = beam_seq.reshape(B, 1, K * T)
    out_shapes = (
        jax.ShapeDtypeStruct((B, 1, 16), jnp.int32),
        jax.ShapeDtypeStruct((B, 1, 8), jnp.float32),
        jax.ShapeDtypeStruct((B, 1, 8), jnp.int32),
    )
    return pl.pallas_call(
        _topk_kernel,
        grid=(B,),
        in_specs=[
            pl.BlockSpec((1, R, C), lambda b: (b, 0, 0)),
            pl.BlockSpec((1, R, 1), lambda b: (b, 0, 0)),
            pl.BlockSpec((1, 1, K * T), lambda b: (b, 0, 0)),
        ],
        out_specs=[
            pl.BlockSpec((1, 1, 16), lambda b: (b, 0, 0)),
            pl.BlockSpec((1, 1, 8), lambda b: (b, 0, 0)),
            pl.BlockSpec((1, 1, 8), lambda b: (b, 0, 0)),
        ],
        out_shape=out_shapes,
        scratch_shapes=[pltpu.VMEM((R, C), jnp.float32)],
    )(lp40, sums40, seq3)


def _pad_tasks(src, dst, n_tiles, k):
    per = src.shape[0] // n_tiles
    src = src.reshape(n_tiles, per)
    dst = dst.reshape(n_tiles, per)
    pad = (-per) % k
    if pad:
        reps = (per + pad + per - 1) // per
        src = jnp.concatenate([src] * reps, axis=1)[:, :per + pad]
        dst = jnp.concatenate([dst] * reps, axis=1)[:, :per + pad]
    rounds = (per + pad) // k
    return src.reshape(n_tiles, rounds, k), dst.reshape(n_tiles, rounds, k)


def _make_sc_gather(n_bsl_rows, n_lp_rows, n_out_rows, w,
                    r_bsl, r_lp, n_st_rows, d_state, r_st, k):
    mesh = plsc.VectorSubcoreMesh(core_axis_name="c", subcore_axis_name="s")

    @functools.partial(
        pl.kernel, mesh=mesh,
        compiler_params=pltpu.CompilerParams(use_tc_tiling_on_sc=False),
        out_type=(
            jax.ShapeDtypeStruct((n_out_rows, w), jnp.float32),
            jax.ShapeDtypeStruct((n_st_rows, d_state), jnp.float32),
        ),
        scratch_types=[
            pltpu.VMEM((r_bsl, k), jnp.int32),
            pltpu.VMEM((r_bsl, k), jnp.int32),
            pltpu.VMEM((r_lp, k), jnp.int32),
            pltpu.VMEM((r_lp, k), jnp.int32),
            pltpu.VMEM((r_st, k), jnp.int32),
            pltpu.VMEM((r_st, k), jnp.int32),
            pltpu.VMEM((k, w), jnp.float32),
            pltpu.VMEM((k, w), jnp.float32),
            pltpu.VMEM((k, d_state), jnp.float32),
            pltpu.SemaphoreType.DMA,
            pltpu.SemaphoreType.DMA,
            pltpu.SemaphoreType.DMA,
            pltpu.SemaphoreType.DMA,
        ],
    )
    def sc_gather(bsl_hbm, lp_hbm, st_hbm,
                  sbsl_hbm, dbsl_hbm, slp_hbm, dlp_hbm, sst_hbm, dst_hbm,
                  out_hbm, stout_hbm,
                  sbsl_v, dbsl_v, slp_v, dlp_v, sst_v, dst_v,
                  buf0, buf1, stbuf, g0, g1, s0, s1):
        nc = plsc.get_sparse_core_info().num_cores
        wid = lax.axis_index("s") * nc + lax.axis_index("c")
        pltpu.sync_copy(sbsl_hbm.at[wid], sbsl_v)
        pltpu.sync_copy(dbsl_hbm.at[wid], dbsl_v)
        pltpu.sync_copy(slp_hbm.at[wid], slp_v)
        pltpu.sync_copy(dlp_hbm.at[wid], dlp_v)
        pltpu.sync_copy(sst_hbm.at[wid], sst_v)
        pltpu.sync_copy(dst_hbm.at[wid], dst_v)

        bufs = (buf0, buf1)
        gsems = (g0, g1)
        ssems = (s0, s1)
        pending = [None, None]

        def run_phase(table, src_v, dst_v_, out, rounds):
            for r in range(rounds):
                p = r & 1
                if pending[p] is not None:
                    pending[p].wait()
                pltpu.async_copy(table.at[src_v.at[r]], bufs[p],
                                 gsems[p]).wait()
                pending[p] = pltpu.async_copy(bufs[p], out.at[dst_v_.at[r]],
                                              ssems[p])

        run_phase(bsl_hbm, sbsl_v, dbsl_v, out_hbm, r_bsl)
        run_phase(lp_hbm, slp_v, dlp_v, out_hbm, r_lp)
        for p in (0, 1):
            if pending[p] is not None:
                pending[p].wait()
        for r in range(r_st):
            pltpu.async_copy(st_hbm.at[sst_v.at[r]], stbuf, g0).wait()
            pltpu.async_copy(stbuf, stout_hbm.at[dst_v.at[r]], s0).wait()

    return sc_gather


def kernel(logprobs, beam_logprobs_sum, beam_seq, beam_seq_logprobs, state,
           beam_size):
    B, K = beam_logprobs_sum.shape
    V = logprobs.shape[-1]
    T = beam_seq.shape[-1]
    S, BK, D = state.shape
    NW = 32
    KROWS = 16
    NC = V // _W

    seq_out, ys_out, bix_out = _run_topk(logprobs, beam_logprobs_sum, beam_seq)
    bix = bix_out[:, 0, :K]

    barange = jnp.arange(B, dtype=jnp.int32)[:, None]
    c_io = jnp.arange(NC, dtype=jnp.int32)
    t_io = jnp.arange(T, dtype=jnp.int32)
    src_beam = barange * K + bix
    dst_beam = barange * K + jnp.arange(K, dtype=jnp.int32)[None, :]
    src_bsl = ((src_beam[:, :, None] * T + t_io[None, None, :])[:, :, :, None]
               * NC + c_io).reshape(-1)
    dst_bsl = ((dst_beam[:, :, None] * (T + 1) + t_io[None, None, :])
               [:, :, :, None] * NC + c_io).reshape(-1)
    src_lp = (src_beam[:, :, None] * NC + c_io).reshape(-1)
    dst_lp = ((dst_beam * (T + 1) + T)[:, :, None] * NC + c_io).reshape(-1)
    s_io = jnp.arange(S, dtype=jnp.int32)[:, None, None] * (B * K)
    src_st = (s_io + src_beam[None]).reshape(-1)
    dst_st = (s_io + dst_beam[None]).reshape(-1)

    sbsl, dbsl = _pad_tasks(src_bsl, dst_bsl, NW, KROWS)
    slp, dlp = _pad_tasks(src_lp, dst_lp, NW, KROWS)
    sst, dst = _pad_tasks(src_st, dst_st, NW, KROWS)

    sc_gather = _make_sc_gather(
        B * K * T * NC, B * K * NC, B * K * (T + 1) * NC, _W,
        sbsl.shape[1], slp.shape[1], S * B * K, D, sst.shape[1], KROWS)
    out_chunks, st_out = sc_gather(
        beam_seq_logprobs.reshape(B * K * T * NC, _W),
        logprobs.reshape(B * K * NC, _W),
        state.reshape(S * B * K, D),
        sbsl, dbsl, slp, dlp, sst, dst)

    new_beam_seq = seq_out[:, 0, :K * (T + 1)].reshape(B, K, T + 1)
    new_beam_logprobs_sum = ys_out[:, 0, :K]
    new_bsl = out_chunks.reshape(B, K, T + 1, V)
    new_state = st_out.reshape(S, B * K, D)
    return (new_beam_seq, new_bsl, new_beam_logprobs_sum, new_state)


# speedup vs baseline: 1.6759x; 1.6759x over previous
import jax
import jax.numpy as jnp
from jax import lax
from jax.experimental import pallas as pl
from jax.experimental.pallas import tpu as pltpu

_NEG_INF = float("-inf")
_BIG = 2147483647
_RPB = 8


def _beam_step_kernel(lp_ref, sums_ref, seq_ref, bsl_ref, st_ref,
                      seq_out_ref, ys_out_ref, out_ref, st_out_ref, scr_ref):
    R = lp_ref.shape[1]
    C = lp_ref.shape[2]
    KT = seq_ref.shape[2]
    K = R // _RPB
    T = KT // K
    cand = lp_ref[0] + sums_ref[0]
    scr_ref[...] = cand
    rowmax = jnp.max(cand, axis=1, keepdims=True)
    riota = lax.broadcasted_iota(jnp.int32, (R, 1), 0)
    ciota = lax.broadcasted_iota(jnp.int32, (1, C), 1)
    i8 = lax.broadcasted_iota(jnp.int32, (1, 8), 1)
    i16 = lax.broadcasted_iota(jnp.int32, (1, 16), 1)
    i10 = lax.broadcasted_iota(jnp.int32, (1, KT), 1)
    seq_row = seq_ref[0]
    ys_row = jnp.zeros((1, 8), jnp.float32)
    seq_out_row = jnp.zeros((1, 16), jnp.int32)
    for k in range(K):
        m = jnp.max(rowmax)
        r = jnp.min(jnp.where(rowmax == m, riota, _BIG))
        row = scr_ref[pl.ds(r, 1), :]
        c = jnp.min(jnp.where(row == m, ciota, _BIG))
        bix = r // _RPB
        six = (r - bix * _RPB) * C + c
        masked = jnp.where(ciota == c, _NEG_INF, row)
        scr_ref[pl.ds(r, 1), :] = masked
        rowmax = jnp.where(riota == r, jnp.max(masked), rowmax)
        ys_row = jnp.where(i8 == k, m, ys_row)
        for t in range(T):
            val = jnp.sum(jnp.where(i10 == bix * T + t, seq_row, 0))
            seq_out_row = jnp.where(i16 == k * (T + 1) + t, val, seq_out_row)
        seq_out_row = jnp.where(i16 == k * (T + 1) + T, six, seq_out_row)
        out_ref[0, pl.ds(k * (T + 1) * _RPB, T * _RPB), :] = (
            bsl_ref[0, pl.ds(bix * T * _RPB, T * _RPB), :])
        out_ref[0, pl.ds((k * (T + 1) + T) * _RPB, _RPB), :] = (
            lp_ref[0, pl.ds(bix * _RPB, _RPB), :])
        st_out_ref[:, 0, pl.ds(k, 1), :] = st_ref[:, 0, pl.ds(bix, 1), :]
    seq_out_ref[0] = seq_out_row
    ys_out_ref[0] = ys_row


def kernel(logprobs, beam_logprobs_sum, beam_seq, beam_seq_logprobs, state,
           beam_size):
    B, K = beam_logprobs_sum.shape
    V = logprobs.shape[-1]
    T = beam_seq.shape[-1]
    S, BK, D = state.shape
    R = K * _RPB
    C = V // _RPB

    lp40 = logprobs.reshape(B, R, C)
    sums40 = jnp.broadcast_to(beam_logprobs_sum[:, :, None],
                              (B, K, _RPB)).reshape(B, R, 1)
    seq3 = beam_seq.reshape(B, 1, K * T)
    bsl80 = beam_seq_logprobs.reshape(B, K * T * _RPB, C)
    st4 = state.reshape(S, B, K, D)

    out_shapes = (
        jax.ShapeDtypeStruct((B, 1, 16), jnp.int32),
        jax.ShapeDtypeStruct((B, 1, 8), jnp.float32),
        jax.ShapeDtypeStruct((B, K * (T + 1) * _RPB, C), jnp.float32),
        jax.ShapeDtypeStruct((S, B, K, D), jnp.float32),
    )
    seq_out, ys_out, out_big, st_out = pl.pallas_call(
        _beam_step_kernel,
        grid=(B,),
        in_specs=[
            pl.BlockSpec((1, R, C), lambda b: (b, 0, 0)),
            pl.BlockSpec((1, R, 1), lambda b: (b, 0, 0)),
            pl.BlockSpec((1, 1, K * T), lambda b: (b, 0, 0)),
            pl.BlockSpec((1, K * T * _RPB, C), lambda b: (b, 0, 0)),
            pl.BlockSpec((S, 1, K, D), lambda b: (0, b, 0, 0)),
        ],
        out_specs=[
            pl.BlockSpec((1, 1, 16), lambda b: (b, 0, 0)),
            pl.BlockSpec((1, 1, 8), lambda b: (b, 0, 0)),
            pl.BlockSpec((1, K * (T + 1) * _RPB, C), lambda b: (b, 0, 0)),
            pl.BlockSpec((S, 1, K, D), lambda b: (0, b, 0, 0)),
        ],
        out_shape=out_shapes,
        scratch_shapes=[pltpu.VMEM((R, C), jnp.float32)],
    )(lp40, sums40, seq3, bsl80, st4)

    new_beam_seq = seq_out[:, 0, :K * (T + 1)].reshape(B, K, T + 1)
    new_beam_logprobs_sum = ys_out[:, 0, :K]
    new_bsl = out_big.reshape(B, K, T + 1, V)
    new_state = st_out.reshape(S, B * K, D)
    return (new_beam_seq, new_bsl, new_beam_logprobs_sum, new_state)


# speedup vs baseline: 3.7036x; 2.2099x over previous
import jax
import jax.numpy as jnp
from jax import lax
from jax.experimental import pallas as pl
from jax.experimental.pallas import tpu as pltpu

_NEG_INF = float("-inf")
_BIG = 2147483647


def _beam_step_kernel(lp_ref, sums_ref, seq_ref, bsl_ref, st_ref,
                      seq_out_ref, ys_out_ref, bsl_out_ref, st_out_ref,
                      scr_ref):
    K = lp_ref.shape[1]
    C = lp_ref.shape[2]
    T = bsl_ref.shape[2]
    cand = lp_ref[0] + sums_ref[0]
    scr_ref[...] = cand
    rowmax = jnp.max(cand, axis=1, keepdims=True)
    riota = lax.broadcasted_iota(jnp.int32, (K, 1), 0)
    ciota = lax.broadcasted_iota(jnp.int32, (1, C), 1)
    i8 = lax.broadcasted_iota(jnp.int32, (1, 8), 1)
    i16 = lax.broadcasted_iota(jnp.int32, (1, 16), 1)
    i10 = lax.broadcasted_iota(jnp.int32, (1, K * T), 1)
    seq_row = seq_ref[0]
    ys_row = jnp.zeros((1, 8), jnp.float32)
    seq_out_row = jnp.zeros((1, 16), jnp.int32)
    for k in range(K):
        m = jnp.max(rowmax)
        bix = jnp.min(jnp.where(rowmax == m, riota, _BIG))
        row = scr_ref[pl.ds(bix, 1), :]
        six = jnp.min(jnp.where(row == m, ciota, _BIG))
        masked = jnp.where(ciota == six, _NEG_INF, row)
        scr_ref[pl.ds(bix, 1), :] = masked
        rowmax = jnp.where(riota == bix, jnp.max(masked), rowmax)
        ys_row = jnp.where(i8 == k, m, ys_row)
        for t in range(T):
            val = jnp.sum(jnp.where(i10 == bix * T + t, seq_row, 0))
            seq_out_row = jnp.where(i16 == k * (T + 1) + t, val, seq_out_row)
        seq_out_row = jnp.where(i16 == k * (T + 1) + T, six, seq_out_row)
        bsl_out_ref[0, pl.ds(k, 1), pl.ds(0, T), :] = (
            bsl_ref[0, pl.ds(bix, 1), :, :])
        bsl_out_ref[0, pl.ds(k, 1), pl.ds(T, 1), :] = (
            lp_ref[pl.ds(0, 1), pl.ds(bix, 1), :])
        st_out_ref[:, 0, pl.ds(k, 1), :] = st_ref[:, 0, pl.ds(bix, 1), :]
    ys_out_ref[0] = ys_row
    seq_out_ref[0] = seq_out_row


def kernel(logprobs, beam_logprobs_sum, beam_seq, beam_seq_logprobs, state,
           beam_size):
    B, K = beam_logprobs_sum.shape
    V = logprobs.shape[-1]
    T = beam_seq.shape[-1]
    S, BK, D = state.shape

    lp3 = logprobs.reshape(B, K, V)
    sums3 = beam_logprobs_sum.reshape(B, K, 1)
    seq3 = beam_seq.reshape(B, 1, K * T)
    st4 = state.reshape(S, B, K, D)

    out_shapes = (
        jax.ShapeDtypeStruct((B, 1, 16), jnp.int32),
        jax.ShapeDtypeStruct((B, 1, 8), jnp.float32),
        jax.ShapeDtypeStruct((B, K, T + 1, V), jnp.float32),
        jax.ShapeDtypeStruct((S, B, K, D), jnp.float32),
    )
    seq_out, ys_out, bsl_out, st_out = pl.pallas_call(
        _beam_step_kernel,
        grid=(B,),
        in_specs=[
            pl.BlockSpec((1, K, V), lambda b: (b, 0, 0)),
            pl.BlockSpec((1, K, 1), lambda b: (b, 0, 0)),
            pl.BlockSpec((1, 1, K * T), lambda b: (b, 0, 0)),
            pl.BlockSpec((1, K, T, V), lambda b: (b, 0, 0, 0)),
            pl.BlockSpec((S, 1, K, D), lambda b: (0, b, 0, 0)),
        ],
        out_specs=[
            pl.BlockSpec((1, 1, 16), lambda b: (b, 0, 0)),
            pl.BlockSpec((1, 1, 8), lambda b: (b, 0, 0)),
            pl.BlockSpec((1, K, T + 1, V), lambda b: (b, 0, 0, 0)),
            pl.BlockSpec((S, 1, K, D), lambda b: (0, b, 0, 0)),
        ],
        out_shape=out_shapes,
        scratch_shapes=[pltpu.VMEM((K, V), jnp.float32)],
    )(lp3, sums3, seq3, beam_seq_logprobs, st4)

    new_beam_seq = seq_out[:, 0, :K * (T + 1)].reshape(B, K, T + 1)
    new_beam_logprobs_sum = ys_out[:, 0, :K]
    new_state = st_out.reshape(S, B * K, D)
    return (new_beam_seq, bsl_out, new_beam_logprobs_sum, new_state)
